# baseline (speedup 1.0000x reference)
import jax
import jax.numpy as jnp
from jax import lax
from jax.experimental import pallas as pl
from jax.experimental.pallas import tpu as pltpu
from jax.experimental.pallas import tpu_sc as plsc

D = 128
B = 16384
NC = 2
NS = 16
L = 16
NW = NC * NS
B_PER_W = B // NW
CHUNK = 128
NCHUNK = B_PER_W // CHUNK
OB = 3


def _body(w_hbm, o_hbm, i_hbm, res_hbm, idx_v, rows_v, out_v,
          semg, semo, semw):
    wid = lax.axis_index("s") * NC + lax.axis_index("c")
    base = wid * B_PER_W
    pltpu.sync_copy(i_hbm.at[wid, 0], idx_v.at[0])
    gathers, outs = [], []
    gathers.append(
        pltpu.async_copy(w_hbm.at[idx_v.at[0]], rows_v.at[0], semg.at[0]))
    pltpu.sync_copy(i_hbm.at[wid, pl.ds(1, NCHUNK - 1)],
                    idx_v.at[pl.ds(1, NCHUNK - 1)])
    outs.append(
        pltpu.async_copy(o_hbm.at[pl.ds(base, CHUNK)],
                         out_v.at[0], semo.at[0]))
    for j in range(1, NCHUNK):
        gathers.append(
            pltpu.async_copy(w_hbm.at[idx_v.at[j]], rows_v.at[j], semg.at[j]))
        if j < OB:
            outs.append(
                pltpu.async_copy(o_hbm.at[pl.ds(base + j * CHUNK, CHUNK)],
                                 out_v.at[j], semo.at[j]))
    writes = []
    for j in range(NCHUNK):
        gathers[j].wait()
        outs[j].wait()

        @plsc.parallel_loop(0, CHUNK, unroll=2)
        def mul_row(r):
            for c in range(D // L):
                s = pl.ds(c * L, L)
                rows_v[j, r, s] = rows_v[j, r, s] * out_v[j % OB, r, s]

        writes.append(
            pltpu.async_copy(rows_v.at[j],
                             res_hbm.at[pl.ds(base + j * CHUNK, CHUNK)],
                             semw))
        if j + OB < NCHUNK:
            outs.append(
                pltpu.async_copy(
                    o_hbm.at[pl.ds(base + (j + OB) * CHUNK, CHUNK)],
                    out_v.at[(j + OB) % OB], semo.at[(j + OB) % OB]))
    for w in writes:
        w.wait()


def kernel(out, taget_adj, weight):
    idx = taget_adj.astype(jnp.int32).reshape(NW, NCHUNK, CHUNK)
    mesh = plsc.VectorSubcoreMesh(core_axis_name="c", subcore_axis_name="s")
    k = pl.kernel(
        _body,
        mesh=mesh,
        out_type=jax.ShapeDtypeStruct((B, D), jnp.float32),
        scratch_types=[
            pltpu.VMEM((NCHUNK, CHUNK), jnp.int32),
            pltpu.VMEM((NCHUNK, CHUNK, D), jnp.float32),
            pltpu.VMEM((OB, CHUNK, D), jnp.float32),
            pltpu.SemaphoreType.DMA((NCHUNK,)),
            pltpu.SemaphoreType.DMA((OB,)),
            pltpu.SemaphoreType.DMA,
        ],
    )
    return k(weight, out, idx)


# speedup vs baseline: 1.0102x; 1.0102x over previous
import jax
import jax.numpy as jnp
from jax import lax
from jax.experimental import pallas as pl
from jax.experimental.pallas import tpu as pltpu
from jax.experimental.pallas import tpu_sc as plsc

D = 128
B = 16384
NC = 2
NS = 16
L = 16
NW = NC * NS
B_PER_W = B // NW
CHUNK = 128
NCHUNK = B_PER_W // CHUNK
OB = 3


def _body(w_hbm, o_hbm, i_hbm, res_hbm, idx_v, rows_v, out_v,
          semg, semo, semw):
    wid = lax.axis_index("s") * NC + lax.axis_index("c")
    base = wid * B_PER_W
    pltpu.sync_copy(i_hbm.at[wid], idx_v)
    gathers, outs = [], []
    for j in range(NCHUNK):
        gathers.append(
            pltpu.async_copy(w_hbm.at[idx_v.at[j]], rows_v.at[j], semg.at[j]))
        if j < OB:
            outs.append(
                pltpu.async_copy(o_hbm.at[pl.ds(base + j * CHUNK, CHUNK)],
                                 out_v.at[j], semo.at[j]))
    writes = []
    for j in range(NCHUNK):
        gathers[j].wait()
        outs[j].wait()

        H = CHUNK // 2
        for h in range(2):
            @plsc.parallel_loop(h * H, (h + 1) * H, unroll=2)
            def mul_row(r):
                for c in range(D // L):
                    s = pl.ds(c * L, L)
                    rows_v[j, r, s] = rows_v[j, r, s] * out_v[j % OB, r, s]

            writes.append(
                pltpu.async_copy(
                    rows_v.at[j, pl.ds(h * H, H)],
                    res_hbm.at[pl.ds(base + j * CHUNK + h * H, H)],
                    semw))
        if j + OB < NCHUNK:
            outs.append(
                pltpu.async_copy(
                    o_hbm.at[pl.ds(base + (j + OB) * CHUNK, CHUNK)],
                    out_v.at[(j + OB) % OB], semo.at[(j + OB) % OB]))
    for w in writes:
        w.wait()


def kernel(out, taget_adj, weight):
    idx = taget_adj.astype(jnp.int32).reshape(NW, NCHUNK, CHUNK)
    mesh = plsc.VectorSubcoreMesh(core_axis_name="c", subcore_axis_name="s")
    k = pl.kernel(
        _body,
        mesh=mesh,
        out_type=jax.ShapeDtypeStruct((B, D), jnp.float32),
        scratch_types=[
            pltpu.VMEM((NCHUNK, CHUNK), jnp.int32),
            pltpu.VMEM((NCHUNK, CHUNK, D), jnp.float32),
            pltpu.VMEM((OB, CHUNK, D), jnp.float32),
            pltpu.SemaphoreType.DMA((NCHUNK,)),
            pltpu.SemaphoreType.DMA((OB,)),
            pltpu.SemaphoreType.DMA,
        ],
    )
    return k(weight, out, idx)


# speedup vs baseline: 1.0477x; 1.0371x over previous
import jax
import jax.numpy as jnp
from jax import lax
from jax.experimental import pallas as pl
from jax.experimental.pallas import tpu as pltpu
from jax.experimental.pallas import tpu_sc as plsc

D = 128
B = 16384
NC = 2
NS = 16
L = 16
NW = NC * NS
B_PER_W = B // NW
CHUNK = 128
NCHUNK = B_PER_W // CHUNK
OB = 3


def _body(w_hbm, o_hbm, i_hbm, res_hbm, idx_v, rows_v, out_v,
          semg, semo, semw):
    wid = lax.axis_index("s") * NC + lax.axis_index("c")
    base = wid * B_PER_W
    pltpu.sync_copy(i_hbm.at[wid], idx_v)

    gathers, outs = [], []
    for j in range(NCHUNK):
        gathers.append(
            pltpu.async_copy(w_hbm.at[idx_v.at[j]], rows_v.at[j], semg.at[j]))
        if j < OB:
            outs.append(
                pltpu.async_copy(o_hbm.at[pl.ds(base + j * CHUNK, CHUNK)],
                                 out_v.at[j], semo.at[j]))
    writes = []
    for j in range(NCHUNK):
        gathers[j].wait()
        outs[j].wait()

        @plsc.parallel_loop(0, CHUNK, unroll=1)
        def mul_row(r):
            for c in range(D // L):
                s = pl.ds(c * L, L)
                rows_v[j, r, s] = rows_v[j, r, s] * out_v[j % OB, r, s]

        writes.append(
            pltpu.async_copy(rows_v.at[j],
                             res_hbm.at[pl.ds(base + j * CHUNK, CHUNK)],
                             semw))
        if j + OB < NCHUNK:
            outs.append(
                pltpu.async_copy(
                    o_hbm.at[pl.ds(base + (j + OB) * CHUNK, CHUNK)],
                    out_v.at[(j + OB) % OB], semo.at[(j + OB) % OB]))
    for w in writes:
        w.wait()


def kernel(out, taget_adj, weight):
    idx = taget_adj.astype(jnp.int32).reshape(NW, NCHUNK, CHUNK)
    mesh = plsc.VectorSubcoreMesh(core_axis_name="c", subcore_axis_name="s")
    k = pl.kernel(
        _body,
        mesh=mesh,
        out_type=jax.ShapeDtypeStruct((B, D), jnp.float32),
        scratch_types=[
            pltpu.VMEM((NCHUNK, CHUNK), jnp.int32),
            pltpu.VMEM((NCHUNK, CHUNK, D), jnp.float32),
            pltpu.VMEM((OB, CHUNK, D), jnp.float32),
            pltpu.SemaphoreType.DMA((NCHUNK,)),
            pltpu.SemaphoreType.DMA((OB,)),
            pltpu.SemaphoreType.DMA,
        ],
    )
    return k(weight, out, idx)
